# baseline (speedup 1.0000x reference)
import jax
import jax.numpy as jnp
from jax.experimental import pallas as pl
from jax.experimental.pallas import tpu as pltpu

B, S, D, H, E, P = 1, 2048, 768, 12, 8, 64
HP = H * P
HE = H * E
EP = E * P
SB = 256
QB = 1024

_SCALE = (1.0 / (P ** 0.5)) ** 0.5


def _proj_route_kernel(x_ref, pw_ref, sw_ref, xp_ref, w_ref):
    x = x_ref[...]
    xb = x.astype(jnp.bfloat16)
    proj = jnp.dot(xb, pw_ref[...], preferred_element_type=jnp.float32)
    xp_ref[...] = (proj * _SCALE).astype(jnp.bfloat16)
    logits = jnp.dot(x, sw_ref[...], preferred_element_type=jnp.float32)
    sel = jax.nn.sigmoid(logits)
    sel_t = sel.T.reshape(H, E, SB)
    eidx = jax.lax.broadcasted_iota(jnp.int32, (H, E, SB), 1)
    m1 = jnp.max(sel_t, axis=1, keepdims=True)
    i1 = jnp.min(jnp.where(sel_t == m1, eidx, E), axis=1, keepdims=True)
    sel2 = jnp.where(eidx == i1, -jnp.inf, sel_t)
    m2 = jnp.max(sel2, axis=1, keepdims=True)
    i2 = jnp.min(jnp.where(sel2 == m2, eidx, E), axis=1, keepdims=True)
    keep = (eidx == i1) | (eidx == i2)
    w_t = jnp.where(keep, sel_t, 0.0)
    w_ref[...] = w_t.reshape(HE, SB).T


def _rep_matrix(n_in, n_out, dtype):
    col = jax.lax.broadcasted_iota(jnp.int32, (n_in, n_out), 1)
    row = jax.lax.broadcasted_iota(jnp.int32, (n_in, n_out), 0)
    return (col // (n_out // n_in) == row).astype(dtype)


def _v_expert_kernel(vsrc_ref, wv_ref, ws_ref, v_ref):
    wv = jnp.transpose(wv_ref[0], (1, 0, 2)).reshape(D, EP)
    inter = jnp.dot(vsrc_ref[...], wv, preferred_element_type=jnp.float32)
    ws = ws_ref[0]
    ws_rep = jnp.dot(ws, _rep_matrix(E, EP, jnp.float32),
                     preferred_element_type=jnp.float32)
    prod = inter * ws_rep
    acc = jnp.zeros((S, P), jnp.float32)
    for e in range(E):
        acc += prod[:, e * P:(e + 1) * P]
    v_ref[0] = acc.astype(jnp.bfloat16)


def _attn_kernel(q_ref, k_ref, v_ref, o_ref):
    for hh in range(2):
        q = q_ref[:, hh * P:(hh + 1) * P]
        k = k_ref[:, hh * P:(hh + 1) * P]
        kf = k.astype(jnp.float32)
        maxkk = jnp.max(jnp.sum(kf * kf, axis=1))
        qf = q.astype(jnp.float32)
        qq = jnp.sum(qf * qf, axis=1, keepdims=True)
        mhat = jnp.sqrt(qq * maxkk) * (1.0 + 2e-3)
        q_aug = jnp.concatenate([q, (-mhat).astype(jnp.bfloat16)], axis=1)
        k_aug = jnp.concatenate([k, jnp.ones((S, 1), jnp.bfloat16)], axis=1)
        logits = jax.lax.dot_general(
            q_aug, k_aug, (((1,), (1,)), ((), ())),
            preferred_element_type=jnp.float32)
        p = jnp.exp(logits).astype(jnp.bfloat16)
        v_aug = jnp.concatenate([v_ref[hh], jnp.ones((S, 1), jnp.bfloat16)],
                                axis=1)
        pv = jnp.dot(p, v_aug, preferred_element_type=jnp.float32)
        r = 1.0 / pv[:, P:P + 1]
        o_ref[:, hh * P:(hh + 1) * P] = (pv[:, :P] * r).astype(jnp.bfloat16)


def _o_expert_kernel(res_ref, ws_ref, rep_ref, wo_ref, out_ref):
    res = res_ref[...]
    ws = ws_ref[...].astype(jnp.bfloat16)
    rep = rep_ref[...]
    acc = jnp.zeros((SB, D), jnp.float32)
    for h in range(H):
        ws_rep = jnp.dot(ws[:, h * E:(h + 1) * E], rep,
                         preferred_element_type=jnp.float32).astype(jnp.bfloat16)
        tmp = pltpu.repeat(res[:, h * P:(h + 1) * P], E, axis=1) * ws_rep
        acc += jnp.dot(tmp, wo_ref[h], preferred_element_type=jnp.float32)
    out_ref[...] = acc


def _proj_route(x, pw_t, sw_t):
    return pl.pallas_call(
        _proj_route_kernel,
        grid=(S // SB,),
        in_specs=[
            pl.BlockSpec((SB, D), lambda i: (i, 0)),
            pl.BlockSpec((D, HP), lambda i: (0, 0)),
            pl.BlockSpec((D, HE), lambda i: (0, 0)),
        ],
        out_specs=[
            pl.BlockSpec((SB, HP), lambda i: (i, 0)),
            pl.BlockSpec((SB, HE), lambda i: (i, 0)),
        ],
        out_shape=[
            jax.ShapeDtypeStruct((S, HP), jnp.bfloat16),
            jax.ShapeDtypeStruct((S, HE), jnp.float32),
        ],
    )(x, pw_t, sw_t)


def kernel(q_src, k_src, v_src, q_w, k_w, Wv, Wo, sel_v, sel_o):
    xq = q_src.reshape(S, D)
    xk = k_src.reshape(S, D)
    xv = v_src.reshape(S, D).astype(jnp.bfloat16)

    qw_t = q_w.T.astype(jnp.bfloat16)
    kw_t = k_w.T.astype(jnp.bfloat16)
    so_t = sel_o.T
    sv_t = sel_v.T

    q, w_o = _proj_route(xq, qw_t, so_t)
    k, w_v = _proj_route(xk, kw_t, sv_t)

    wv_4 = Wv.reshape(H, E, D, P).astype(jnp.bfloat16)
    wsel_v = w_v.reshape(S, H, E).transpose(1, 0, 2)

    v = pl.pallas_call(
        _v_expert_kernel,
        grid=(H,),
        in_specs=[
            pl.BlockSpec((S, D), lambda h: (0, 0)),
            pl.BlockSpec((1, E, D, P), lambda h: (h, 0, 0, 0)),
            pl.BlockSpec((1, S, E), lambda h: (h, 0, 0)),
        ],
        out_specs=pl.BlockSpec((1, S, P), lambda h: (h, 0, 0)),
        out_shape=jax.ShapeDtypeStruct((H, S, P), jnp.bfloat16),
    )(xv, wv_4, wsel_v)

    res = pl.pallas_call(
        _attn_kernel,
        grid=(H // 2, S // QB),
        in_specs=[
            pl.BlockSpec((QB, 2 * P), lambda g, i: (i, g)),
            pl.BlockSpec((S, 2 * P), lambda g, i: (0, g)),
            pl.BlockSpec((2, S, P), lambda g, i: (g, 0, 0)),
        ],
        out_specs=pl.BlockSpec((QB, 2 * P), lambda g, i: (i, g)),
        out_shape=jax.ShapeDtypeStruct((S, HP), jnp.bfloat16),
    )(q, k, v)

    wo_r = Wo.reshape(H, EP, D).astype(jnp.bfloat16)
    col = jax.lax.broadcasted_iota(jnp.int32, (E, EP), 1)
    row = jax.lax.broadcasted_iota(jnp.int32, (E, EP), 0)
    rep_o = (row == col // P).astype(jnp.bfloat16)

    out = pl.pallas_call(
        _o_expert_kernel,
        grid=(S // SB,),
        in_specs=[
            pl.BlockSpec((SB, HP), lambda i: (i, 0)),
            pl.BlockSpec((SB, HE), lambda i: (i, 0)),
            pl.BlockSpec((E, EP), lambda i: (0, 0)),
            pl.BlockSpec((H, EP, D), lambda i: (0, 0, 0)),
        ],
        out_specs=pl.BlockSpec((SB, D), lambda i: (i, 0)),
        out_shape=jax.ShapeDtypeStruct((S, D), jnp.float32),
    )(res, w_o, rep_o, wo_r)

    return out.reshape(B, S, D)


# speedup vs baseline: 1.1652x; 1.1652x over previous
import jax
import jax.numpy as jnp
from jax.experimental import pallas as pl
from jax.experimental.pallas import tpu as pltpu

B, S, D, H, E, P = 1, 2048, 768, 12, 8, 64
HP = H * P
HE = H * E
EP = E * P
SB = 256

_SCALE = (1.0 / (P ** 0.5)) ** 0.5


def _proj_route_kernel(x_ref, pw_ref, sw_ref, xp_ref, w_ref):
    x = x_ref[...]
    xb = x.astype(jnp.bfloat16)
    pw = pw_ref[...].astype(jnp.bfloat16)
    proj = jax.lax.dot_general(xb, pw, (((1,), (1,)), ((), ())),
                               preferred_element_type=jnp.float32)
    xp_ref[...] = (proj * _SCALE).astype(jnp.bfloat16)
    logits = jax.lax.dot_general(x, sw_ref[...], (((1,), (1,)), ((), ())),
                                 preferred_element_type=jnp.float32)
    sel = jax.nn.sigmoid(logits)
    sel_t = sel.T.reshape(H, E, SB)
    eidx = jax.lax.broadcasted_iota(jnp.int32, (H, E, SB), 1)
    m1 = jnp.max(sel_t, axis=1, keepdims=True)
    i1 = jnp.min(jnp.where(sel_t == m1, eidx, E), axis=1, keepdims=True)
    sel2 = jnp.where(eidx == i1, -jnp.inf, sel_t)
    m2 = jnp.max(sel2, axis=1, keepdims=True)
    i2 = jnp.min(jnp.where(sel2 == m2, eidx, E), axis=1, keepdims=True)
    keep = (eidx == i1) | (eidx == i2)
    w_t = jnp.where(keep, sel_t, 0.0)
    w_ref[...] = w_t.reshape(HE, SB).T


def _rep_matrix(n_in, n_out, dtype):
    col = jax.lax.broadcasted_iota(jnp.int32, (n_in, n_out), 1)
    row = jax.lax.broadcasted_iota(jnp.int32, (n_in, n_out), 0)
    return (col // (n_out // n_in) == row).astype(dtype)


def _vattn_kernel(xv_ref, wv_ref, ws_ref, q_ref, k_ref, wo_ref,
                  o_ref, wob_ref):
    xvb = xv_ref[...].astype(jnp.bfloat16)
    rep = _rep_matrix(E, EP, jnp.float32)
    for hh in range(2):
        wv = jnp.transpose(wv_ref[hh], (1, 0, 2)).reshape(D, EP)
        inter = jnp.dot(xvb, wv.astype(jnp.bfloat16),
                        preferred_element_type=jnp.float32)
        ws_rep = jnp.dot(ws_ref[hh], rep, preferred_element_type=jnp.float32)
        prod = inter * ws_rep
        acc = jnp.zeros((S, P), jnp.float32)
        for e in range(E):
            acc += prod[:, e * P:(e + 1) * P]
        v = acc.astype(jnp.bfloat16)
        q = q_ref[:, hh * P:(hh + 1) * P]
        k = k_ref[:, hh * P:(hh + 1) * P]
        kf = k.astype(jnp.float32)
        maxkk = jnp.max(jnp.sum(kf * kf, axis=1))
        qf = q.astype(jnp.float32)
        qq = jnp.sum(qf * qf, axis=1, keepdims=True)
        mhat = jnp.sqrt(qq * maxkk) * (1.0 + 2e-3)
        q_aug = jnp.concatenate([q, (-mhat).astype(jnp.bfloat16)], axis=1)
        k_aug = jnp.concatenate([k, jnp.ones((S, 1), jnp.bfloat16)], axis=1)
        logits = jax.lax.dot_general(
            q_aug, k_aug, (((1,), (1,)), ((), ())),
            preferred_element_type=jnp.float32)
        p = jnp.exp(logits).astype(jnp.bfloat16)
        v_aug = jnp.concatenate([v, jnp.ones((S, 1), jnp.bfloat16)], axis=1)
        pv = jnp.dot(p, v_aug, preferred_element_type=jnp.float32)
        r = 1.0 / pv[:, P:P + 1]
        o_ref[:, hh * P:(hh + 1) * P] = (pv[:, :P] * r).astype(jnp.bfloat16)
    wob_ref[...] = wo_ref[...].astype(jnp.bfloat16)


def _o_expert_kernel(res_ref, ws_ref, wo_ref, out_ref):
    res = res_ref[...]
    ws = ws_ref[...].astype(jnp.bfloat16)
    rep = _rep_matrix(E, EP, jnp.bfloat16)
    acc = jnp.zeros((SB, D), jnp.float32)
    for h in range(H):
        ws_rep = jnp.dot(ws[:, h * E:(h + 1) * E], rep,
                         preferred_element_type=jnp.float32).astype(jnp.bfloat16)
        tmp = pltpu.repeat(res[:, h * P:(h + 1) * P], E, axis=1) * ws_rep
        acc += jnp.dot(tmp, wo_ref[h], preferred_element_type=jnp.float32)
    out_ref[...] = acc


def _proj_route(x, pw, sw):
    return pl.pallas_call(
        _proj_route_kernel,
        grid=(S // SB,),
        in_specs=[
            pl.BlockSpec((SB, D), lambda i: (i, 0)),
            pl.BlockSpec((HP, D), lambda i: (0, 0)),
            pl.BlockSpec((HE, D), lambda i: (0, 0)),
        ],
        out_specs=[
            pl.BlockSpec((SB, HP), lambda i: (i, 0)),
            pl.BlockSpec((SB, HE), lambda i: (i, 0)),
        ],
        out_shape=[
            jax.ShapeDtypeStruct((S, HP), jnp.bfloat16),
            jax.ShapeDtypeStruct((S, HE), jnp.float32),
        ],
    )(x, pw, sw)


def kernel(q_src, k_src, v_src, q_w, k_w, Wv, Wo, sel_v, sel_o):
    xq = q_src.reshape(S, D)
    xk = k_src.reshape(S, D)
    xv = v_src.reshape(S, D)

    q, w_o = _proj_route(xq, q_w, sel_o)
    k, w_v = _proj_route(xk, k_w, sel_v)

    wv_4 = Wv.reshape(H, E, D, P)
    wo_3 = Wo.reshape(H, EP, D)
    wsel_v = w_v.reshape(S, H, E).transpose(1, 0, 2)

    res, wo_bf = pl.pallas_call(
        _vattn_kernel,
        grid=(H // 2,),
        in_specs=[
            pl.BlockSpec((S, D), lambda g: (0, 0)),
            pl.BlockSpec((2, E, D, P), lambda g: (g, 0, 0, 0)),
            pl.BlockSpec((2, S, E), lambda g: (g, 0, 0)),
            pl.BlockSpec((S, 2 * P), lambda g: (0, g)),
            pl.BlockSpec((S, 2 * P), lambda g: (0, g)),
            pl.BlockSpec((2, EP, D), lambda g: (g, 0, 0)),
        ],
        out_specs=[
            pl.BlockSpec((S, 2 * P), lambda g: (0, g)),
            pl.BlockSpec((2, EP, D), lambda g: (g, 0, 0)),
        ],
        out_shape=[
            jax.ShapeDtypeStruct((S, HP), jnp.bfloat16),
            jax.ShapeDtypeStruct((H, EP, D), jnp.bfloat16),
        ],
    )(xv, wv_4, wsel_v, q, k, wo_3)

    out = pl.pallas_call(
        _o_expert_kernel,
        grid=(S // SB,),
        in_specs=[
            pl.BlockSpec((SB, HP), lambda i: (i, 0)),
            pl.BlockSpec((SB, HE), lambda i: (i, 0)),
            pl.BlockSpec((H, EP, D), lambda i: (0, 0, 0)),
        ],
        out_specs=pl.BlockSpec((SB, D), lambda i: (i, 0)),
        out_shape=jax.ShapeDtypeStruct((S, D), jnp.float32),
    )(res, w_o, wo_bf)

    return out.reshape(B, S, D)
